# baseline (speedup 1.0000x reference)
import functools

import jax
import jax.numpy as jnp
import numpy as np
from jax.experimental import pallas as pl
from jax.experimental.pallas import tpu as pltpu
from jax.experimental.pallas import tpu_sc as plsc

_MASK_RATIO = float(np.random.default_rng(0).uniform(0.1, 0.5))

_CB = 24


def _build_mask_row(sites, w):
    n_sites = sites.shape[1]
    mesh = plsc.ScalarSubcoreMesh(axis_name="c", num_cores=2)

    @pl.kernel(
        out_type=jax.ShapeDtypeStruct((1, w), jnp.float32),
        mesh=mesh,
        scratch_types=[
            pltpu.SMEM((w,), jnp.float32),
            pltpu.SMEM((n_sites,), jnp.int32),
            pltpu.SemaphoreType.DMA,
        ],
    )
    def mask_kernel(sites_ref, o_ref, row_ref, idx_ref, sem):
        core = jax.lax.axis_index("c")

        @pl.when(core == 0)
        def _():
            pltpu.async_copy(sites_ref.at[0], idx_ref, sem).wait()

            @pl.loop(0, w)
            def _(i):
                row_ref[i] = jnp.float32(1)

            @pl.loop(0, n_sites)
            def _(i):
                row_ref[idx_ref[i]] = jnp.float32(0)

            pltpu.async_copy(row_ref, o_ref.at[0], sem).wait()

    return mask_kernel(sites)


def _mask_mul_kernel(mask_ref, x_ref, o_ref, *, h, w):
    mask2d = jnp.broadcast_to(mask_ref[...], (h, w))
    o_ref[...] = x_ref[...] * mask2d[None, None]


def kernel(x):
    b, c, h, w = x.shape
    n_sites = int(_MASK_RATIO * w)
    perm = jax.random.permutation(jax.random.key(1), w)
    sites = perm[:n_sites].astype(jnp.int32).reshape(1, n_sites)

    maskrow = _build_mask_row(sites, w)

    return pl.pallas_call(
        functools.partial(_mask_mul_kernel, h=h, w=w),
        grid=(b, c // _CB),
        in_specs=[
            pl.BlockSpec((1, w), lambda i, j: (0, 0)),
            pl.BlockSpec((1, _CB, h, w), lambda i, j: (i, j, 0, 0)),
        ],
        out_specs=pl.BlockSpec((1, _CB, h, w), lambda i, j: (i, j, 0, 0)),
        out_shape=jax.ShapeDtypeStruct((b, c, h, w), x.dtype),
        compiler_params=pltpu.CompilerParams(
            dimension_semantics=("arbitrary", "arbitrary"),
        ),
    )(maskrow, x)


# speedup vs baseline: 1.0622x; 1.0622x over previous
import functools

import jax
import jax.numpy as jnp
import numpy as np
from jax.experimental import pallas as pl
from jax.experimental.pallas import tpu as pltpu

_MASK_RATIO = float(np.random.default_rng(0).uniform(0.1, 0.5))

_BLK = 2048


def _mask_mul_kernel(sites_ref, x_ref, o_ref, *, n_sites, h, w):
    sites = sites_ref[0, :].reshape(n_sites, 1)
    cols = jax.lax.broadcasted_iota(jnp.int32, (1, w), 1)
    hit = jnp.any(sites == cols, axis=0, keepdims=True)
    maskrow = jnp.where(hit, jnp.float32(0), jnp.float32(1))
    mask2d = jnp.broadcast_to(maskrow, (h, w))
    o_ref[...] = x_ref[...] * mask2d[None, None]


_CB = 24


def kernel(x):
    b, c, h, w = x.shape
    n_sites = int(_MASK_RATIO * w)
    perm = jax.random.permutation(jax.random.key(1), w)
    sites = perm[:n_sites].astype(jnp.int32).reshape(1, n_sites)

    return pl.pallas_call(
        functools.partial(_mask_mul_kernel, n_sites=n_sites, h=h, w=w),
        grid=(b, c // _CB),
        in_specs=[
            pl.BlockSpec((1, n_sites), lambda i, j: (0, 0)),
            pl.BlockSpec((1, _CB, h, w), lambda i, j: (i, j, 0, 0)),
        ],
        out_specs=pl.BlockSpec((1, _CB, h, w), lambda i, j: (i, j, 0, 0)),
        out_shape=jax.ShapeDtypeStruct((b, c, h, w), x.dtype),
        compiler_params=pltpu.CompilerParams(
            dimension_semantics=("arbitrary", "arbitrary"),
        ),
    )(sites, x)


# speedup vs baseline: 1.0687x; 1.0061x over previous
import functools

import jax
import jax.numpy as jnp
import numpy as np
from jax.experimental import pallas as pl
from jax.experimental.pallas import tpu as pltpu

_MASK_RATIO = float(np.random.default_rng(0).uniform(0.1, 0.5))

_NSTEP = 24
_NBUF = 3
_CHUNK = 1024


def _manual_body(sites_ref, x_hbm, o_hbm, b0, b1, b2, sin, sout, *,
                 n_sites, w, blkr):
    bufs = (b0, b1, b2)

    sites = sites_ref[0, :].reshape(n_sites, 1)
    cols = jax.lax.broadcasted_iota(jnp.int32, (1, w), 1)
    hit = jnp.any(sites == cols, axis=0, keepdims=True)
    maskrow = jnp.where(hit, jnp.float32(0), jnp.float32(1))

    def in_copy(step, slot):
        return pltpu.make_async_copy(
            x_hbm.at[pl.ds(step * blkr, blkr), :], bufs[slot], sin.at[slot])

    def out_copy(step, slot):
        return pltpu.make_async_copy(
            bufs[slot], o_hbm.at[pl.ds(step * blkr, blkr), :], sout.at[slot])

    def compute(slot):
        buf = bufs[slot]

        def body(r, carry):
            blk = buf[pl.ds(r * _CHUNK, _CHUNK), :]
            mask2d = jnp.broadcast_to(maskrow, (_CHUNK, w))
            buf[pl.ds(r * _CHUNK, _CHUNK), :] = blk * mask2d
            return carry

        jax.lax.fori_loop(0, blkr // _CHUNK, body, 0)

    for k in range(_NBUF):
        in_copy(k, k).start()
    for i in range(_NSTEP):
        s = i % _NBUF
        in_copy(i, s).wait()
        compute(s)
        out_copy(i, s).start()
        if i + _NBUF < _NSTEP:
            out_copy(i, s).wait()
            in_copy(i + _NBUF, s).start()
    for i in range(_NSTEP - _NBUF, _NSTEP):
        out_copy(i, i % _NBUF).wait()


def kernel(x):
    b, c, h, w = x.shape
    n_sites = int(_MASK_RATIO * w)
    perm = jax.random.permutation(jax.random.key(1), w)
    sites = perm[:n_sites].astype(jnp.int32).reshape(1, n_sites)

    rows = b * c * h
    blkr = rows // _NSTEP
    x2 = x.reshape(rows, w)

    out = pl.pallas_call(
        functools.partial(_manual_body, n_sites=n_sites, w=w, blkr=blkr),
        in_specs=[
            pl.BlockSpec((1, n_sites), lambda: (0, 0)),
            pl.BlockSpec(memory_space=pltpu.MemorySpace.HBM),
        ],
        out_specs=pl.BlockSpec(memory_space=pltpu.MemorySpace.HBM),
        out_shape=jax.ShapeDtypeStruct((rows, w), x.dtype),
        scratch_shapes=[
            pltpu.VMEM((blkr, w), jnp.float32),
            pltpu.VMEM((blkr, w), jnp.float32),
            pltpu.VMEM((blkr, w), jnp.float32),
            pltpu.SemaphoreType.DMA((_NBUF,)),
            pltpu.SemaphoreType.DMA((_NBUF,)),
        ],
    )(sites, x2)
    return out.reshape(b, c, h, w)
